# baseline (speedup 1.0000x reference)
import functools

import jax
import jax.numpy as jnp
from jax import lax
from jax.experimental import pallas as pl
from jax.experimental.pallas import tpu as pltpu
from jax.experimental.pallas import tpu_sc as plsc

NC = 2
NS = 16
NW = NC * NS
BLK = 128
G = 2
RI = 4
TC_ROWS = 1000


def _sc_gather_scatter(table, gdidx, npad, h):
  k = gdidx.shape[1]
  ng = k // RI
  rows_per_sub = npad // NS
  chunks = rows_per_sub // BLK

  def body(table_ref, gdidx_ref, parts_ref,
           agg, islot, buf0, buf1,
           sg0, sg1, si0, si1, si2, si3):
    bufs = (buf0, buf1)
    sg = (sg0, sg1)
    si = (si0, si1, si2, si3)
    c = lax.axis_index("c")
    s = lax.axis_index("s")
    w = s * NC + c

    def zb(q, carry):
      buf0[q // (h // 16), pl.ds((q % (h // 16)) * 16, 16)] = (
          jnp.zeros((16,), jnp.float32))
      return carry
    lax.fori_loop(0, BLK * (h // 16), zb, 0)
    for t in range(chunks):
      pltpu.sync_copy(buf0, agg.at[pl.ds(s * rows_per_sub + t * BLK, BLK)])

    for t in range(RI):
      pltpu.async_copy(gdidx_ref.at[w, t], islot.at[t], si[t])
    for b in range(G):
      pltpu.make_async_copy(gdidx_ref.at[w, b], islot.at[b], si[b]).wait()
      pltpu.async_copy(table_ref.at[islot.at[b, 0]], bufs[b], sg[b])
    plsc.subcore_barrier()

    def grp(o, carry):
      j0 = o * RI
      for u in range(RI):
        j = j0 + u
        b = u % G
        pltpu.make_async_copy(table_ref.at[islot.at[u, 0]], bufs[b],
                              sg[b]).wait()
        pltpu.sync_copy(bufs[b], agg.at[islot.at[u, 1]], add=True)

        @pl.when(j + RI < k)
        def _():
          pltpu.async_copy(gdidx_ref.at[w, j + RI], islot.at[u], si[u])

        @pl.when(j + G < k)
        def _():
          u2 = (u + G) % RI
          pltpu.make_async_copy(gdidx_ref.at[w, 0], islot.at[u2],
                                si[u2]).wait()
          pltpu.async_copy(table_ref.at[islot.at[u2, 0]], bufs[b], sg[b])
      return carry
    lax.fori_loop(0, ng, grp, 0)
    plsc.subcore_barrier()

    for t in range(chunks):
      r0 = s * rows_per_sub + t * BLK
      pltpu.sync_copy(agg.at[pl.ds(r0, BLK)], buf0)
      pltpu.sync_copy(buf0, parts_ref.at[c, pl.ds(r0, BLK)])

  mesh = plsc.VectorSubcoreMesh(core_axis_name="c", subcore_axis_name="s")
  return pl.kernel(
      body,
      out_type=jax.ShapeDtypeStruct((NC, npad, h), jnp.float32),
      mesh=mesh,
      scratch_types=[
          pltpu.VMEM_SHARED((npad, h), jnp.float32),
          pltpu.VMEM((RI, 2, BLK), jnp.int32),
          pltpu.VMEM((BLK, h), jnp.float32),
          pltpu.VMEM((BLK, h), jnp.float32),
          pltpu.SemaphoreType.DMA,
          pltpu.SemaphoreType.DMA,
          pltpu.SemaphoreType.DMA,
          pltpu.SemaphoreType.DMA,
          pltpu.SemaphoreType.DMA,
          pltpu.SemaphoreType.DMA,
      ],
  )(table, gdidx)


def _proj_body(r, x_ref, w_ref, loop_ref, b_ref, xp_ref, sl_ref):
  xb = x_ref[...]
  for i in range(r):
    xp_ref[i] = jnp.dot(xb, w_ref[i], preferred_element_type=jnp.float32)
  sl_ref[...] = (jnp.dot(xb, loop_ref[...], preferred_element_type=jnp.float32)
                 + b_ref[...])


def _mid_body(r, p_ref, sl_ref, w_ref, loop_ref, b_ref, xp_ref, sl1_ref):
  hb = jnp.maximum(p_ref[0] + p_ref[1] + sl_ref[...], 0.0)
  for i in range(r):
    xp_ref[i] = jnp.dot(hb, w_ref[i], preferred_element_type=jnp.float32)
  sl1_ref[...] = (jnp.dot(hb, loop_ref[...],
                          preferred_element_type=jnp.float32) + b_ref[...])


def _fin_body(p_ref, sl_ref, out_ref):
  out_ref[...] = p_ref[0] + p_ref[1] + sl_ref[...]


def kernel(x, edge_index, edge_type, W0, b0, loop0, W1, b1, loop1):
  n, d = x.shape
  e = edge_type.shape[0]
  r, _, h = W0.shape
  assert n % TC_ROWS == 0
  grid = n // TC_ROWS

  epad = -(-e // (NW * BLK * RI)) * (NW * BLK * RI)
  k = epad // (NW * BLK)
  npad = -(-(n + 1) // (NS * BLK)) * (NS * BLK)

  src = edge_index[0].astype(jnp.int32)
  dst = edge_index[1].astype(jnp.int32)
  et = edge_type.astype(jnp.int32)
  pad = epad - e
  src = jnp.concatenate([src, jnp.zeros((pad,), jnp.int32)])
  et = jnp.concatenate([et, jnp.zeros((pad,), jnp.int32)])
  dst = jnp.concatenate([dst, jnp.full((pad,), n, jnp.int32)])
  gidx = (et * n + src).reshape(NW, k, 1, BLK)
  didx = dst.reshape(NW, k, 1, BLK)
  gdidx = jnp.concatenate([gidx, didx], axis=2)

  wfull = pl.BlockSpec((r, d, h), lambda i: (0, 0, 0))
  lfull = pl.BlockSpec((d, h), lambda i: (0, 0))
  bfull = pl.BlockSpec((1, h), lambda i: (0, 0))
  rowblk = pl.BlockSpec((TC_ROWS, d), lambda i: (i, 0))
  xpblk = pl.BlockSpec((r, TC_ROWS, h), lambda i: (0, i, 0))

  proj = pl.pallas_call(
      functools.partial(_proj_body, r),
      grid=(grid,),
      in_specs=[rowblk, wfull, lfull, bfull],
      out_specs=[xpblk, rowblk],
      out_shape=[jax.ShapeDtypeStruct((r, n, h), jnp.float32),
                 jax.ShapeDtypeStruct((n, h), jnp.float32)],
  )
  mid = pl.pallas_call(
      functools.partial(_mid_body, r),
      grid=(grid,),
      in_specs=[pl.BlockSpec((NC, TC_ROWS, h), lambda i: (0, i, 0)),
                rowblk, wfull, lfull, bfull],
      out_specs=[xpblk, rowblk],
      out_shape=[jax.ShapeDtypeStruct((r, n, h), jnp.float32),
                 jax.ShapeDtypeStruct((n, h), jnp.float32)],
  )
  fin = pl.pallas_call(
      _fin_body,
      grid=(grid,),
      in_specs=[pl.BlockSpec((NC, TC_ROWS, h), lambda i: (0, i, 0)), rowblk],
      out_specs=rowblk,
      out_shape=jax.ShapeDtypeStruct((n, h), jnp.float32),
  )

  b0r = b0.reshape(1, h)
  b1r = b1.reshape(1, h)

  xp0, sl0 = proj(x, W0, loop0, b0r)
  parts0 = _sc_gather_scatter(xp0.reshape(r * n, h), gdidx, npad, h)
  xp1, sl1 = mid(parts0, sl0, W1, loop1, b1r)
  parts1 = _sc_gather_scatter(xp1.reshape(r * n, h), gdidx, npad, h)
  return fin(parts1, sl1)


# speedup vs baseline: 1.0487x; 1.0487x over previous
import functools

import jax
import jax.numpy as jnp
from jax import lax
from jax.experimental import pallas as pl
from jax.experimental.pallas import tpu as pltpu
from jax.experimental.pallas import tpu_sc as plsc

NC = 2
NS = 16
NW = NC * NS
BLK = 128
G = 2
RI = 4
TC_ROWS = 1000


DBITS = 14


def _sc_gather_scatter(table, pidx, npad, h):
  k = pidx.shape[1]
  ng = k // G
  rows_per_sub = npad // NS
  chunks = rows_per_sub // BLK
  nv = BLK // 16

  def body(table_ref, pidx_ref, parts_ref,
           agg, pidx_v, gscr, dscr, buf0, buf1, sg0, sg1):
    bufs = (buf0, buf1)
    sg = (sg0, sg1)
    c = lax.axis_index("c")
    s = lax.axis_index("s")
    w = s * NC + c

    def unpack_g(j, slot):
      row = pidx_v.at[j]
      for t in range(nv):
        gscr[slot, pl.ds(t * 16, 16)] = (
            lax.shift_right_logical(row[pl.ds(t * 16, 16)], DBITS))

    def unpack_d(j):
      row = pidx_v.at[j]
      for t in range(nv):
        dscr[0, pl.ds(t * 16, 16)] = row[pl.ds(t * 16, 16)] & ((1 << DBITS) - 1)

    def zb(q, carry):
      for t in range(h // 16):
        buf0[q, pl.ds(t * 16, 16)] = jnp.zeros((16,), jnp.float32)
      return carry
    lax.fori_loop(0, BLK, zb, 0)
    for t in range(chunks):
      pltpu.sync_copy(buf0, agg.at[pl.ds(s * rows_per_sub + t * BLK, BLK)])

    pltpu.sync_copy(pidx_ref.at[w], pidx_v)
    for b in range(G):
      unpack_g(b, b)
      pltpu.async_copy(table_ref.at[gscr.at[b]], bufs[b], sg[b])
    plsc.subcore_barrier()

    def grp(o, carry):
      j0 = o * G
      for b in range(G):
        j = j0 + b
        pltpu.make_async_copy(table_ref.at[gscr.at[b]], bufs[b],
                              sg[b]).wait()
        unpack_d(j)
        pltpu.sync_copy(bufs[b], agg.at[dscr.at[0]], add=True)

        @pl.when(j + G < k)
        def _():
          unpack_g(j + G, b)
          pltpu.async_copy(table_ref.at[gscr.at[b]], bufs[b], sg[b])
      return carry
    lax.fori_loop(0, ng, grp, 0)
    plsc.subcore_barrier()

    for t in range(chunks):
      r0 = s * rows_per_sub + t * BLK
      pltpu.sync_copy(agg.at[pl.ds(r0, BLK)], buf0)
      pltpu.sync_copy(buf0, parts_ref.at[c, pl.ds(r0, BLK)])

  mesh = plsc.VectorSubcoreMesh(core_axis_name="c", subcore_axis_name="s")
  return pl.kernel(
      body,
      out_type=jax.ShapeDtypeStruct((NC, npad, h), jnp.float32),
      mesh=mesh,
      scratch_types=[
          pltpu.VMEM_SHARED((npad, h), jnp.float32),
          pltpu.VMEM((k, BLK), jnp.int32),
          pltpu.VMEM((G, BLK), jnp.int32),
          pltpu.VMEM((1, BLK), jnp.int32),
          pltpu.VMEM((BLK, h), jnp.float32),
          pltpu.VMEM((BLK, h), jnp.float32),
          pltpu.SemaphoreType.DMA,
          pltpu.SemaphoreType.DMA,
      ],
  )(table, pidx)


def _proj_body(r, x_ref, w_ref, loop_ref, b_ref, xp_ref, sl_ref):
  xb = x_ref[...]
  for i in range(r):
    xp_ref[i] = jnp.dot(xb, w_ref[i], preferred_element_type=jnp.float32)
  sl_ref[...] = (jnp.dot(xb, loop_ref[...], preferred_element_type=jnp.float32)
                 + b_ref[...])


def _mid_body(r, p_ref, sl_ref, w_ref, loop_ref, b_ref, xp_ref, sl1_ref):
  hb = jnp.maximum(p_ref[0] + p_ref[1] + sl_ref[...], 0.0)
  for i in range(r):
    xp_ref[i] = jnp.dot(hb, w_ref[i], preferred_element_type=jnp.float32)
  sl1_ref[...] = (jnp.dot(hb, loop_ref[...],
                          preferred_element_type=jnp.float32) + b_ref[...])


def _fin_body(p_ref, sl_ref, out_ref):
  out_ref[...] = p_ref[0] + p_ref[1] + sl_ref[...]


def kernel(x, edge_index, edge_type, W0, b0, loop0, W1, b1, loop1):
  n, d = x.shape
  e = edge_type.shape[0]
  r, _, h = W0.shape
  assert n % TC_ROWS == 0
  grid = n // TC_ROWS

  epad = -(-e // (NW * BLK * G)) * (NW * BLK * G)
  k = epad // (NW * BLK)
  npad = -(-(n + 1) // (NS * BLK)) * (NS * BLK)
  assert npad <= (1 << DBITS)

  src = edge_index[0].astype(jnp.int32)
  dst = edge_index[1].astype(jnp.int32)
  et = edge_type.astype(jnp.int32)
  pad = epad - e
  src = jnp.concatenate([src, jnp.zeros((pad,), jnp.int32)])
  et = jnp.concatenate([et, jnp.zeros((pad,), jnp.int32)])
  dst = jnp.concatenate([dst, jnp.full((pad,), n, jnp.int32)])
  pidx = (((et * n + src) << DBITS) | dst).reshape(NW, k, BLK)

  wfull = pl.BlockSpec((r, d, h), lambda i: (0, 0, 0))
  lfull = pl.BlockSpec((d, h), lambda i: (0, 0))
  bfull = pl.BlockSpec((1, h), lambda i: (0, 0))
  rowblk = pl.BlockSpec((TC_ROWS, d), lambda i: (i, 0))
  xpblk = pl.BlockSpec((r, TC_ROWS, h), lambda i: (0, i, 0))

  proj = pl.pallas_call(
      functools.partial(_proj_body, r),
      grid=(grid,),
      in_specs=[rowblk, wfull, lfull, bfull],
      out_specs=[xpblk, rowblk],
      out_shape=[jax.ShapeDtypeStruct((r, n, h), jnp.float32),
                 jax.ShapeDtypeStruct((n, h), jnp.float32)],
  )
  mid = pl.pallas_call(
      functools.partial(_mid_body, r),
      grid=(grid,),
      in_specs=[pl.BlockSpec((NC, TC_ROWS, h), lambda i: (0, i, 0)),
                rowblk, wfull, lfull, bfull],
      out_specs=[xpblk, rowblk],
      out_shape=[jax.ShapeDtypeStruct((r, n, h), jnp.float32),
                 jax.ShapeDtypeStruct((n, h), jnp.float32)],
  )
  fin = pl.pallas_call(
      _fin_body,
      grid=(grid,),
      in_specs=[pl.BlockSpec((NC, TC_ROWS, h), lambda i: (0, i, 0)), rowblk],
      out_specs=rowblk,
      out_shape=jax.ShapeDtypeStruct((n, h), jnp.float32),
  )

  b0r = b0.reshape(1, h)
  b1r = b1.reshape(1, h)

  xp0, sl0 = proj(x, W0, loop0, b0r)
  parts0 = _sc_gather_scatter(xp0.reshape(r * n, h), pidx, npad, h)
  xp1, sl1 = mid(parts0, sl0, W1, loop1, b1r)
  parts1 = _sc_gather_scatter(xp1.reshape(r * n, h), pidx, npad, h)
  return fin(parts1, sl1)


# speedup vs baseline: 1.2468x; 1.1889x over previous
import functools

import jax
import jax.numpy as jnp
from jax import lax
from jax.experimental import pallas as pl
from jax.experimental.pallas import tpu as pltpu
from jax.experimental.pallas import tpu_sc as plsc

NC = 2
NS = 16
NW = NC * NS
BR = 1
BLK = BR * 128
TC_ROWS = 1000
DBITS = 14


def _sc_gather_scatter(table, pidx, npad, h):
  k = pidx.shape[1]
  rows_per_sub = npad // NS
  nv = BLK // 16

  def body(table_ref, pidx_ref, parts_ref,
           agg, pidx_v, gscr, dscr, buf, sg):
    c = lax.axis_index("c")
    s = lax.axis_index("s")
    w = s * NC + c

    def unpack_g(j):
      row = pidx_v.at[j]
      for t in range(nv):
        gscr[t // (nv // BR), pl.ds((t % (nv // BR)) * 16, 16)] = (
            lax.shift_right_logical(row[t // (nv // BR),
                                        pl.ds((t % (nv // BR)) * 16, 16)],
                                    DBITS))

    def unpack_d(j):
      row = pidx_v.at[j]
      for t in range(nv):
        dscr[t // (nv // BR), pl.ds((t % (nv // BR)) * 16, 16)] = (
            row[t // (nv // BR), pl.ds((t % (nv // BR)) * 16, 16)]
            & ((1 << DBITS) - 1))

    def zb(q, carry):
      for t in range(h // 16):
        buf[q, pl.ds(t * 16, 16)] = jnp.zeros((16,), jnp.float32)
      return carry
    lax.fori_loop(0, BLK, zb, 0)
    r0 = 0
    while r0 < rows_per_sub:
      rr = min(BLK, rows_per_sub - r0)
      pltpu.sync_copy(buf.at[pl.ds(0, rr)],
                      agg.at[pl.ds(s * rows_per_sub + r0, rr)])
      r0 += rr

    pltpu.sync_copy(pidx_ref.at[w], pidx_v)
    plsc.subcore_barrier()

    def blk(j, carry):
      unpack_g(j)
      cp = pltpu.async_copy(table_ref.at[gscr.at[0]], buf, sg)
      unpack_d(j)
      cp.wait()
      pltpu.sync_copy(buf, agg.at[dscr.at[0]], add=True)
      return carry
    lax.fori_loop(0, k, blk, 0)
    plsc.subcore_barrier()

    r0 = 0
    while r0 < rows_per_sub:
      rr = min(BLK, rows_per_sub - r0)
      rbase = s * rows_per_sub + r0
      pltpu.sync_copy(agg.at[pl.ds(rbase, rr)], buf.at[pl.ds(0, rr)])
      pltpu.sync_copy(buf.at[pl.ds(0, rr)], parts_ref.at[c, pl.ds(rbase, rr)])
      r0 += rr

  mesh = plsc.VectorSubcoreMesh(core_axis_name="c", subcore_axis_name="s")
  return pl.kernel(
      body,
      out_type=jax.ShapeDtypeStruct((NC, npad, h), jnp.float32),
      mesh=mesh,
      scratch_types=[
          pltpu.VMEM_SHARED((npad, h), jnp.float32),
          pltpu.VMEM((k, BR, 128), jnp.int32),
          pltpu.VMEM((BR, 128), jnp.int32),
          pltpu.VMEM((BR, 128), jnp.int32),
          pltpu.VMEM((BLK, h), jnp.float32),
          pltpu.SemaphoreType.DMA,
      ],
  )(table, pidx)


def _proj_body(r, x_ref, w_ref, loop_ref, b_ref, xp_ref, sl_ref):
  xb = x_ref[...]
  for i in range(r):
    xp_ref[i] = jnp.dot(xb, w_ref[i], preferred_element_type=jnp.float32)
  sl_ref[...] = (jnp.dot(xb, loop_ref[...], preferred_element_type=jnp.float32)
                 + b_ref[...])


def _mid_body(r, p_ref, sl_ref, w_ref, loop_ref, b_ref, xp_ref, sl1_ref):
  hb = jnp.maximum(p_ref[0] + p_ref[1] + sl_ref[...], 0.0)
  for i in range(r):
    xp_ref[i] = jnp.dot(hb, w_ref[i], preferred_element_type=jnp.float32)
  sl1_ref[...] = (jnp.dot(hb, loop_ref[...],
                          preferred_element_type=jnp.float32) + b_ref[...])


def _fin_body(p_ref, sl_ref, out_ref):
  out_ref[...] = p_ref[0] + p_ref[1] + sl_ref[...]


def kernel(x, edge_index, edge_type, W0, b0, loop0, W1, b1, loop1):
  n, d = x.shape
  e = edge_type.shape[0]
  r, _, h = W0.shape
  assert n % TC_ROWS == 0
  grid = n // TC_ROWS

  epad = -(-e // (NW * BLK)) * (NW * BLK)
  k = epad // (NW * BLK)
  npad = -(-(n + 1) // (NS * 8)) * (NS * 8)
  assert npad <= (1 << DBITS)

  src = edge_index[0].astype(jnp.int32)
  dst = edge_index[1].astype(jnp.int32)
  et = edge_type.astype(jnp.int32)
  pad = epad - e
  src = jnp.concatenate([src, jnp.zeros((pad,), jnp.int32)])
  et = jnp.concatenate([et, jnp.zeros((pad,), jnp.int32)])
  dst = jnp.concatenate([dst, jnp.full((pad,), n, jnp.int32)])
  pidx = (((et * n + src) << DBITS) | dst).reshape(NW, k, BR, 128)

  wfull = pl.BlockSpec((r, d, h), lambda i: (0, 0, 0))
  lfull = pl.BlockSpec((d, h), lambda i: (0, 0))
  bfull = pl.BlockSpec((1, h), lambda i: (0, 0))
  rowblk = pl.BlockSpec((TC_ROWS, d), lambda i: (i, 0))
  xpblk = pl.BlockSpec((r, TC_ROWS, h), lambda i: (0, i, 0))

  proj = pl.pallas_call(
      functools.partial(_proj_body, r),
      grid=(grid,),
      in_specs=[rowblk, wfull, lfull, bfull],
      out_specs=[xpblk, rowblk],
      out_shape=[jax.ShapeDtypeStruct((r, n, h), jnp.float32),
                 jax.ShapeDtypeStruct((n, h), jnp.float32)],
  )
  mid = pl.pallas_call(
      functools.partial(_mid_body, r),
      grid=(grid,),
      in_specs=[pl.BlockSpec((NC, TC_ROWS, h), lambda i: (0, i, 0)),
                rowblk, wfull, lfull, bfull],
      out_specs=[xpblk, rowblk],
      out_shape=[jax.ShapeDtypeStruct((r, n, h), jnp.float32),
                 jax.ShapeDtypeStruct((n, h), jnp.float32)],
  )
  fin = pl.pallas_call(
      _fin_body,
      grid=(grid,),
      in_specs=[pl.BlockSpec((NC, TC_ROWS, h), lambda i: (0, i, 0)), rowblk],
      out_specs=rowblk,
      out_shape=jax.ShapeDtypeStruct((n, h), jnp.float32),
  )

  b0r = b0.reshape(1, h)
  b1r = b1.reshape(1, h)

  xp0, sl0 = proj(x, W0, loop0, b0r)
  parts0 = _sc_gather_scatter(xp0.reshape(r * n, h), pidx, npad, h)
  xp1, sl1 = mid(parts0, sl0, W1, loop1, b1r)
  parts1 = _sc_gather_scatter(xp1.reshape(r * n, h), pidx, npad, h)
  return fin(parts1, sl1)
